# baseline (speedup 1.0000x reference)
import functools

import numpy as np
import jax
import jax.numpy as jnp
from jax import lax
from jax.experimental import pallas as pl
from jax.experimental.pallas import tpu as pltpu
from jax.experimental.pallas import tpu_sc as plsc

N = 10000
E = 320000
D = 128
HID = 128
OUT = 64
NGRAPH = 128

NCORES = 2
NSUB = 16
NTILES = NCORES * NSUB
EPT = E // NTILES
CHUNK = 128
CHUNKS = 80
STAGES = 2
SCHUNKS = CHUNKS // STAGES
EPT_PAD = CHUNKS * CHUNK
AGG_ROWS = 10112
ZROWS = AGG_ROWS // NSUB

_BN_C = 1.0 / np.sqrt(1.0 + 1e-5)

BLK = 1000
NBLK = N // BLK


def _sc_body(edges, h, out, agg, src_buf, dst_buf, rows, rows1, sem, sem1):
    cid = lax.axis_index("c")
    sid = lax.axis_index("s")
    wid = cid * NSUB + sid

    def _zrow(r, c):
        for k in range(8):
            rows[r, pl.ds(k * 16, 16)] = jnp.zeros((16,), jnp.float32)
        return c

    lax.fori_loop(0, CHUNK, _zrow, 0)
    zbase = sid * ZROWS
    pltpu.sync_copy(rows, agg.at[pl.ds(zbase, 128)])
    pltpu.sync_copy(rows, agg.at[pl.ds(zbase + 128, 128)])
    pltpu.sync_copy(rows, agg.at[pl.ds(zbase + 256, 128)])
    pltpu.sync_copy(rows, agg.at[pl.ds(zbase + 384, 128)])
    pltpu.sync_copy(rows.at[pl.ds(0, ZROWS - 512)],
                    agg.at[pl.ds(zbase + 512, ZROWS - 512)])
    plsc.subcore_barrier()

    last = SCHUNKS - 1
    for s in range(STAGES):
        pltpu.sync_copy(edges.at[0, wid, s], src_buf)
        pltpu.sync_copy(edges.at[1, wid, s], dst_buf)
        pltpu.async_copy(h.at[src_buf.at[0]], rows, sem)
        pltpu.async_copy(h.at[src_buf.at[1]], rows1, sem1)

        def _edge_pair(t, c):
            j0 = 2 * t
            pltpu.make_async_copy(h.at[src_buf.at[j0]], rows, sem).wait()
            pltpu.sync_copy(rows, agg.at[dst_buf.at[j0]], add=True)
            pltpu.async_copy(h.at[src_buf.at[jnp.minimum(j0 + 2, last)]],
                             rows, sem)
            j1 = j0 + 1
            pltpu.make_async_copy(h.at[src_buf.at[j1]], rows1, sem1).wait()
            pltpu.sync_copy(rows1, agg.at[dst_buf.at[j1]], add=True)
            pltpu.async_copy(h.at[src_buf.at[jnp.minimum(j1 + 2, last)]],
                             rows1, sem1)
            return c

        lax.fori_loop(0, SCHUNKS // 2, _edge_pair, 0)
        pltpu.make_async_copy(h.at[src_buf.at[last]], rows, sem).wait()
        pltpu.make_async_copy(h.at[src_buf.at[last]], rows1, sem1).wait()
    plsc.subcore_barrier()

    obase = sid * ZROWS
    pltpu.sync_copy(agg.at[pl.ds(obase, ZROWS)],
                    out.at[cid, pl.ds(obase, ZROWS)])


_sc_scatter = pl.kernel(
    _sc_body,
    out_type=jax.ShapeDtypeStruct((NCORES, AGG_ROWS, D), jnp.float32),
    mesh=plsc.VectorSubcoreMesh(core_axis_name="c", subcore_axis_name="s",
                                num_cores=NCORES, num_subcores=NSUB),
    scratch_types=[
        pltpu.VMEM_SHARED((AGG_ROWS, D), jnp.float32),
        pltpu.VMEM((SCHUNKS, CHUNK), jnp.int32),
        pltpu.VMEM((SCHUNKS, CHUNK), jnp.int32),
        pltpu.VMEM((CHUNK, D), jnp.float32),
        pltpu.VMEM((CHUNK, D), jnp.float32),
        pltpu.SemaphoreType.DMA,
        pltpu.SemaphoreType.DMA,
    ],
)


def _mlp_math(eps_ref, h_ref, p0_ref, p1_ref, w1_ref, b1_ref, w2_ref, b2_ref):
    z = h_ref[...] * eps_ref[0, 0] + p0_ref[...] + p1_ref[...]
    z = jnp.dot(z, w1_ref[...], preferred_element_type=jnp.float32) + b1_ref[...]
    z = jnp.maximum(z, 0.0)
    z = jnp.dot(z, w2_ref[...], preferred_element_type=jnp.float32) + b2_ref[...]
    return jnp.maximum(z, 0.0)


def _mlp_body(eps_ref, h_ref, p0_ref, p1_ref, w1_ref, b1_ref, w2_ref, b2_ref,
              out_ref):
    out_ref[...] = _mlp_math(eps_ref, h_ref, p0_ref, p1_ref, w1_ref, b1_ref,
                             w2_ref, b2_ref)


def _mlp_pool_body(eps_ref, h_ref, p0_ref, p1_ref, w1_ref, b1_ref, w2_ref,
                   b2_ref, batch_ref, pooled_ref):
    hn = _mlp_math(eps_ref, h_ref, p0_ref, p1_ref, w1_ref, b1_ref, w2_ref,
                   b2_ref)
    seg = lax.broadcasted_iota(jnp.int32, (NGRAPH, BLK), 0)
    mask = (seg == batch_ref[0, 0, :][None, :]).astype(jnp.float32)
    contrib = jnp.dot(mask, hn, preferred_element_type=jnp.float32)
    i = pl.program_id(0)

    @pl.when(i == 0)
    def _():
        pooled_ref[...] = contrib

    @pl.when(i > 0)
    def _():
        pooled_ref[...] += contrib


def _head_body(pooled_ref, w1_ref, b1_ref, w2_ref, b2_ref, out_ref):
    z = jnp.dot(pooled_ref[...], w1_ref[...],
                preferred_element_type=jnp.float32) + b1_ref[...]
    z = jnp.maximum(z, 0.0)
    z = jnp.dot(z, w2_ref[...], preferred_element_type=jnp.float32) + b2_ref[...]
    m = jnp.max(z, axis=-1, keepdims=True)
    lse = jnp.log(jnp.sum(jnp.exp(z - m), axis=-1, keepdims=True)) + m
    out_ref[...] = z - lse


def _row_spec(i):
    return (i, 0)


def _fixed_spec(i):
    return (0, 0)


_COMMON_SPECS = [
    pl.BlockSpec(memory_space=pltpu.SMEM),
    pl.BlockSpec((BLK, D), _row_spec),
    pl.BlockSpec((BLK, D), _row_spec),
    pl.BlockSpec((BLK, D), _row_spec),
    pl.BlockSpec((D, 2 * HID), _fixed_spec),
    pl.BlockSpec((1, 2 * HID), _fixed_spec),
    pl.BlockSpec((2 * HID, HID), _fixed_spec),
    pl.BlockSpec((1, HID), _fixed_spec),
]

_mlp = pl.pallas_call(
    _mlp_body,
    grid=(NBLK,),
    in_specs=_COMMON_SPECS,
    out_specs=pl.BlockSpec((BLK, D), _row_spec),
    out_shape=jax.ShapeDtypeStruct((N, HID), jnp.float32),
)

_mlp_pool = pl.pallas_call(
    _mlp_pool_body,
    grid=(NBLK,),
    in_specs=_COMMON_SPECS + [pl.BlockSpec((1, 1, BLK), lambda i: (i, 0, 0))],
    out_specs=pl.BlockSpec((NGRAPH, HID), _fixed_spec),
    out_shape=jax.ShapeDtypeStruct((NGRAPH, HID), jnp.float32),
)

_head = pl.pallas_call(
    _head_body,
    out_shape=jax.ShapeDtypeStruct((NGRAPH, OUT), jnp.float32),
)


def kernel(x, edge_index, batch, params):
    pad = EPT_PAD - EPT
    src = jnp.concatenate(
        [edge_index[0].reshape(NTILES, EPT),
         jnp.zeros((NTILES, pad), jnp.int32)], axis=1)
    dst = jnp.concatenate(
        [edge_index[1].reshape(NTILES, EPT),
         jnp.full((NTILES, pad), N, jnp.int32)], axis=1)
    edges = jnp.stack([src, dst]).reshape(2, NTILES, STAGES, SCHUNKS, CHUNK)
    batch_r = batch.reshape(NBLK, 1, BLK)

    h = x
    nlayers = len(params["layers"])
    pooled = None
    for li, lp in enumerate(params["layers"]):
        a1 = lp["g1"] * _BN_C
        w1 = lp["W1"] * a1[None, :]
        b1 = (lp["b1"] * a1 + lp["be1"]).reshape(1, 2 * HID)
        a2 = lp["g2"] * _BN_C
        w2 = lp["W2"] * a2[None, :]
        b2 = (lp["b2"] * a2 + lp["be2"]).reshape(1, HID)
        onepe = (1.0 + lp["eps"]).astype(jnp.float32).reshape(1, 1)
        part = _sc_scatter(edges, h)
        if li < nlayers - 1:
            h = _mlp(onepe, h, part[0], part[1], w1, b1, w2, b2)
        else:
            pooled = _mlp_pool(onepe, h, part[0], part[1], w1, b1, w2, b2,
                               batch_r)

    a3 = params["g3"] * _BN_C
    l1w = params["lin1_W"] * a3[None, :]
    l1b = (params["lin1_b"] * a3 + params["be3"]).reshape(1, HID)
    l2b = params["lin2_b"].reshape(1, OUT)
    return _head(pooled, l1w, l1b, params["lin2_W"], l2b)


# speedup vs baseline: 1.2257x; 1.2257x over previous
import functools

import numpy as np
import jax
import jax.numpy as jnp
from jax import lax
from jax.experimental import pallas as pl
from jax.experimental.pallas import tpu as pltpu
from jax.experimental.pallas import tpu_sc as plsc

N = 10000
E = 320000
D = 128
HID = 128
OUT = 64
NGRAPH = 128

NCORES = 2
NSUB = 16
NTILES = NCORES * NSUB
EPT = E // NTILES
CHUNK = 128
GRP = 1
CHUNKS = 80
STAGES = 2
SCHUNKS = CHUNKS // STAGES
SGROUPS = SCHUNKS // GRP
EPT_PAD = CHUNKS * CHUNK
AGG_ROWS = 10112
ZROWS = AGG_ROWS // NSUB

_BN_C = 1.0 / np.sqrt(1.0 + 1e-5)

BLK = 1000
NBLK = N // BLK


def _sc_body(edges, h, out, agg, src_buf, dst_buf, rows, rows1, sem, sem1):
    cid = lax.axis_index("c")
    sid = lax.axis_index("s")
    wid = cid * NSUB + sid

    def _zrow(r, c):
        for k in range(8):
            rows[r, pl.ds(k * 16, 16)] = jnp.zeros((16,), jnp.float32)
        return c

    lax.fori_loop(0, CHUNK, _zrow, 0)
    zbase = sid * ZROWS
    pltpu.sync_copy(rows, agg.at[pl.ds(zbase, 128)])
    pltpu.sync_copy(rows, agg.at[pl.ds(zbase + 128, 128)])
    pltpu.sync_copy(rows, agg.at[pl.ds(zbase + 256, 128)])
    pltpu.sync_copy(rows, agg.at[pl.ds(zbase + 384, 128)])
    pltpu.sync_copy(rows.at[pl.ds(0, ZROWS - 512)],
                    agg.at[pl.ds(zbase + 512, ZROWS - 512)])
    plsc.subcore_barrier()

    last = SCHUNKS - 1
    for s in range(STAGES):
        pltpu.sync_copy(edges.at[0, wid, s], src_buf)
        pltpu.sync_copy(edges.at[1, wid, s], dst_buf)
        pltpu.async_copy(h.at[src_buf.at[0]], rows, sem).wait()

        def _edge_pair(t, c):
            j0 = 2 * t
            cp1 = pltpu.async_copy(h.at[src_buf.at[j0 + 1]], rows1, sem1)
            pltpu.sync_copy(rows, agg.at[dst_buf.at[j0]], add=True)
            cp1.wait()
            cp0 = pltpu.async_copy(
                h.at[src_buf.at[jnp.minimum(j0 + 2, last)]], rows, sem)
            pltpu.sync_copy(rows1, agg.at[dst_buf.at[j0 + 1]], add=True)
            cp0.wait()
            return c

        lax.fori_loop(0, SCHUNKS // 2, _edge_pair, 0)
    plsc.subcore_barrier()

    obase = sid * ZROWS
    pltpu.sync_copy(agg.at[pl.ds(obase, ZROWS)],
                    out.at[cid, pl.ds(obase, ZROWS)])


_sc_scatter = pl.kernel(
    _sc_body,
    out_type=jax.ShapeDtypeStruct((NCORES, AGG_ROWS, D), jnp.float32),
    mesh=plsc.VectorSubcoreMesh(core_axis_name="c", subcore_axis_name="s",
                                num_cores=NCORES, num_subcores=NSUB),
    scratch_types=[
        pltpu.VMEM_SHARED((AGG_ROWS, D), jnp.float32),
        pltpu.VMEM((SCHUNKS, CHUNK), jnp.int32),
        pltpu.VMEM((SCHUNKS, CHUNK), jnp.int32),
        pltpu.VMEM((CHUNK, D), jnp.float32),
        pltpu.VMEM((CHUNK, D), jnp.float32),
        pltpu.SemaphoreType.DMA,
        pltpu.SemaphoreType.DMA,
    ],
)


def _mlp_math(eps_ref, h_ref, p0_ref, p1_ref, w1_ref, b1_ref, w2_ref, b2_ref):
    z = h_ref[...] * eps_ref[0, 0] + p0_ref[...] + p1_ref[...]
    z = jnp.dot(z, w1_ref[...], preferred_element_type=jnp.float32) + b1_ref[...]
    z = jnp.maximum(z, 0.0)
    z = jnp.dot(z, w2_ref[...], preferred_element_type=jnp.float32) + b2_ref[...]
    return jnp.maximum(z, 0.0)


def _mlp_body(eps_ref, h_ref, p0_ref, p1_ref, w1_ref, b1_ref, w2_ref, b2_ref,
              out_ref):
    out_ref[...] = _mlp_math(eps_ref, h_ref, p0_ref, p1_ref, w1_ref, b1_ref,
                             w2_ref, b2_ref)


def _mlp_pool_body(eps_ref, h_ref, p0_ref, p1_ref, w1_ref, b1_ref, w2_ref,
                   b2_ref, batch_ref, pooled_ref):
    hn = _mlp_math(eps_ref, h_ref, p0_ref, p1_ref, w1_ref, b1_ref, w2_ref,
                   b2_ref)
    seg = lax.broadcasted_iota(jnp.int32, (NGRAPH, BLK), 0)
    mask = (seg == batch_ref[0, 0, :][None, :]).astype(jnp.float32)
    contrib = jnp.dot(mask, hn, preferred_element_type=jnp.float32)
    i = pl.program_id(0)

    @pl.when(i == 0)
    def _():
        pooled_ref[...] = contrib

    @pl.when(i > 0)
    def _():
        pooled_ref[...] += contrib


def _head_body(pooled_ref, w1_ref, b1_ref, w2_ref, b2_ref, out_ref):
    z = jnp.dot(pooled_ref[...], w1_ref[...],
                preferred_element_type=jnp.float32) + b1_ref[...]
    z = jnp.maximum(z, 0.0)
    z = jnp.dot(z, w2_ref[...], preferred_element_type=jnp.float32) + b2_ref[...]
    m = jnp.max(z, axis=-1, keepdims=True)
    lse = jnp.log(jnp.sum(jnp.exp(z - m), axis=-1, keepdims=True)) + m
    out_ref[...] = z - lse


def _row_spec(i):
    return (i, 0)


def _fixed_spec(i):
    return (0, 0)


_COMMON_SPECS = [
    pl.BlockSpec(memory_space=pltpu.SMEM),
    pl.BlockSpec((BLK, D), _row_spec),
    pl.BlockSpec((BLK, D), _row_spec),
    pl.BlockSpec((BLK, D), _row_spec),
    pl.BlockSpec((D, 2 * HID), _fixed_spec),
    pl.BlockSpec((1, 2 * HID), _fixed_spec),
    pl.BlockSpec((2 * HID, HID), _fixed_spec),
    pl.BlockSpec((1, HID), _fixed_spec),
]

_mlp = pl.pallas_call(
    _mlp_body,
    grid=(NBLK,),
    in_specs=_COMMON_SPECS,
    out_specs=pl.BlockSpec((BLK, D), _row_spec),
    out_shape=jax.ShapeDtypeStruct((N, HID), jnp.float32),
)

_mlp_pool = pl.pallas_call(
    _mlp_pool_body,
    grid=(NBLK,),
    in_specs=_COMMON_SPECS + [pl.BlockSpec((1, 1, BLK), lambda i: (i, 0, 0))],
    out_specs=pl.BlockSpec((NGRAPH, HID), _fixed_spec),
    out_shape=jax.ShapeDtypeStruct((NGRAPH, HID), jnp.float32),
)

_head = pl.pallas_call(
    _head_body,
    out_shape=jax.ShapeDtypeStruct((NGRAPH, OUT), jnp.float32),
)


def kernel(x, edge_index, batch, params):
    pad = EPT_PAD - EPT
    src = jnp.concatenate(
        [edge_index[0].reshape(NTILES, EPT),
         jnp.zeros((NTILES, pad), jnp.int32)], axis=1)
    dst = jnp.concatenate(
        [edge_index[1].reshape(NTILES, EPT),
         jnp.full((NTILES, pad), N, jnp.int32)], axis=1)
    edges = jnp.stack([src, dst]).reshape(2, NTILES, STAGES, SCHUNKS, CHUNK)
    batch_r = batch.reshape(NBLK, 1, BLK)

    h = x
    nlayers = len(params["layers"])
    pooled = None
    for li, lp in enumerate(params["layers"]):
        a1 = lp["g1"] * _BN_C
        w1 = lp["W1"] * a1[None, :]
        b1 = (lp["b1"] * a1 + lp["be1"]).reshape(1, 2 * HID)
        a2 = lp["g2"] * _BN_C
        w2 = lp["W2"] * a2[None, :]
        b2 = (lp["b2"] * a2 + lp["be2"]).reshape(1, HID)
        onepe = (1.0 + lp["eps"]).astype(jnp.float32).reshape(1, 1)
        part = _sc_scatter(edges, h)
        if li < nlayers - 1:
            h = _mlp(onepe, h, part[0], part[1], w1, b1, w2, b2)
        else:
            pooled = _mlp_pool(onepe, h, part[0], part[1], w1, b1, w2, b2,
                               batch_r)

    a3 = params["g3"] * _BN_C
    l1w = params["lin1_W"] * a3[None, :]
    l1b = (params["lin1_b"] * a3 + params["be3"]).reshape(1, HID)
    l2b = params["lin2_b"].reshape(1, OUT)
    return _head(pooled, l1w, l1b, params["lin2_W"], l2b)


# speedup vs baseline: 1.5069x; 1.2294x over previous
import functools

import numpy as np
import jax
import jax.numpy as jnp
from jax import lax
from jax.experimental import pallas as pl
from jax.experimental.pallas import tpu as pltpu
from jax.experimental.pallas import tpu_sc as plsc

N = 10000
E = 320000
D = 128
HID = 128
OUT = 64
NGRAPH = 128

NCORES = 2
NSUB = 16
NTILES = NCORES * NSUB
EPT = E // NTILES
CHUNK = 128
GRP = 1
CHUNKS = 80
STAGES = 2
SCHUNKS = CHUNKS // STAGES
SGROUPS = SCHUNKS // GRP
EPT_PAD = CHUNKS * CHUNK
AGG_ROWS = 10112
ZROWS = AGG_ROWS // NSUB

_BN_C = 1.0 / np.sqrt(1.0 + 1e-5)

BLK = 1000
NBLK = N // BLK


def _sc_body(edges, h, out, agg, src_buf, dst_buf, rows, rows1, sem, sem1):
    cid = lax.axis_index("c")
    sid = lax.axis_index("s")
    wid = cid * NSUB + sid

    def _zrow(r, c):
        for k in range(8):
            rows[r, pl.ds(k * 16, 16)] = jnp.zeros((16,), jnp.float32)
        return c

    lax.fori_loop(0, CHUNK, _zrow, 0)
    zbase = sid * ZROWS
    pltpu.sync_copy(rows, agg.at[pl.ds(zbase, 128)])
    pltpu.sync_copy(rows, agg.at[pl.ds(zbase + 128, 128)])
    pltpu.sync_copy(rows, agg.at[pl.ds(zbase + 256, 128)])
    pltpu.sync_copy(rows, agg.at[pl.ds(zbase + 384, 128)])
    pltpu.sync_copy(rows.at[pl.ds(0, ZROWS - 512)],
                    agg.at[pl.ds(zbase + 512, ZROWS - 512)])
    plsc.subcore_barrier()

    for s in range(STAGES):
        with jax.named_scope("sc_stage_idx"):
            pltpu.sync_copy(edges.at[0, wid, s], src_buf)
            pltpu.sync_copy(edges.at[1, wid, s], dst_buf)

        def _edge_chunk(j, c):
            with jax.named_scope("sc_gather"):
                pltpu.async_copy(h.at[src_buf.at[j]], rows, sem).wait()
            with jax.named_scope("sc_scatter"):
                pltpu.sync_copy(rows, agg.at[dst_buf.at[j]], add=True)
            return c

        lax.fori_loop(0, SCHUNKS, _edge_chunk, 0)
    plsc.subcore_barrier()

    obase = sid * ZROWS
    pltpu.sync_copy(agg.at[pl.ds(obase, ZROWS)],
                    out.at[cid, pl.ds(obase, ZROWS)])


_sc_scatter = pl.kernel(
    _sc_body,
    out_type=jax.ShapeDtypeStruct((NCORES, AGG_ROWS, D), jnp.float32),
    mesh=plsc.VectorSubcoreMesh(core_axis_name="c", subcore_axis_name="s",
                                num_cores=NCORES, num_subcores=NSUB),
    scratch_types=[
        pltpu.VMEM_SHARED((AGG_ROWS, D), jnp.float32),
        pltpu.VMEM((SCHUNKS, CHUNK), jnp.int32),
        pltpu.VMEM((SCHUNKS, CHUNK), jnp.int32),
        pltpu.VMEM((CHUNK, D), jnp.float32),
        pltpu.VMEM((CHUNK, D), jnp.float32),
        pltpu.SemaphoreType.DMA,
        pltpu.SemaphoreType.DMA,
    ],
)


def _mlp_math(eps_ref, h_ref, p0_ref, p1_ref, w1_ref, b1_ref, w2_ref, b2_ref):
    z = h_ref[...] * eps_ref[0, 0] + p0_ref[...] + p1_ref[...]
    z = jnp.dot(z, w1_ref[...], preferred_element_type=jnp.float32) + b1_ref[...]
    z = jnp.maximum(z, 0.0)
    z = jnp.dot(z, w2_ref[...], preferred_element_type=jnp.float32) + b2_ref[...]
    return jnp.maximum(z, 0.0)


def _mlp_body(eps_ref, h_ref, p0_ref, p1_ref, w1_ref, b1_ref, w2_ref, b2_ref,
              out_ref):
    out_ref[...] = _mlp_math(eps_ref, h_ref, p0_ref, p1_ref, w1_ref, b1_ref,
                             w2_ref, b2_ref)


def _mlp_pool_body(eps_ref, h_ref, p0_ref, p1_ref, w1_ref, b1_ref, w2_ref,
                   b2_ref, batch_ref, pooled_ref):
    hn = _mlp_math(eps_ref, h_ref, p0_ref, p1_ref, w1_ref, b1_ref, w2_ref,
                   b2_ref)
    seg = lax.broadcasted_iota(jnp.int32, (NGRAPH, BLK), 0)
    mask = (seg == batch_ref[0, 0, :][None, :]).astype(jnp.float32)
    contrib = jnp.dot(mask, hn, preferred_element_type=jnp.float32)
    i = pl.program_id(0)

    @pl.when(i == 0)
    def _():
        pooled_ref[...] = contrib

    @pl.when(i > 0)
    def _():
        pooled_ref[...] += contrib


def _head_body(pooled_ref, w1_ref, b1_ref, w2_ref, b2_ref, out_ref):
    z = jnp.dot(pooled_ref[...], w1_ref[...],
                preferred_element_type=jnp.float32) + b1_ref[...]
    z = jnp.maximum(z, 0.0)
    z = jnp.dot(z, w2_ref[...], preferred_element_type=jnp.float32) + b2_ref[...]
    m = jnp.max(z, axis=-1, keepdims=True)
    lse = jnp.log(jnp.sum(jnp.exp(z - m), axis=-1, keepdims=True)) + m
    out_ref[...] = z - lse


def _row_spec(i):
    return (i, 0)


def _fixed_spec(i):
    return (0, 0)


_COMMON_SPECS = [
    pl.BlockSpec(memory_space=pltpu.SMEM),
    pl.BlockSpec((BLK, D), _row_spec),
    pl.BlockSpec((BLK, D), _row_spec),
    pl.BlockSpec((BLK, D), _row_spec),
    pl.BlockSpec((D, 2 * HID), _fixed_spec),
    pl.BlockSpec((1, 2 * HID), _fixed_spec),
    pl.BlockSpec((2 * HID, HID), _fixed_spec),
    pl.BlockSpec((1, HID), _fixed_spec),
]

_mlp = pl.pallas_call(
    _mlp_body,
    grid=(NBLK,),
    in_specs=_COMMON_SPECS,
    out_specs=pl.BlockSpec((BLK, D), _row_spec),
    out_shape=jax.ShapeDtypeStruct((N, HID), jnp.float32),
)

_mlp_pool = pl.pallas_call(
    _mlp_pool_body,
    grid=(NBLK,),
    in_specs=_COMMON_SPECS + [pl.BlockSpec((1, 1, BLK), lambda i: (i, 0, 0))],
    out_specs=pl.BlockSpec((NGRAPH, HID), _fixed_spec),
    out_shape=jax.ShapeDtypeStruct((NGRAPH, HID), jnp.float32),
)

_head = pl.pallas_call(
    _head_body,
    out_shape=jax.ShapeDtypeStruct((NGRAPH, OUT), jnp.float32),
)


def kernel(x, edge_index, batch, params):
    pad = EPT_PAD - EPT
    src = jnp.concatenate(
        [edge_index[0].reshape(NTILES, EPT),
         jnp.zeros((NTILES, pad), jnp.int32)], axis=1)
    dst = jnp.concatenate(
        [edge_index[1].reshape(NTILES, EPT),
         jnp.full((NTILES, pad), N, jnp.int32)], axis=1)
    edges = jnp.stack([src, dst]).reshape(2, NTILES, STAGES, SCHUNKS, CHUNK)
    batch_r = batch.reshape(NBLK, 1, BLK)

    h = x
    nlayers = len(params["layers"])
    pooled = None
    for li, lp in enumerate(params["layers"]):
        a1 = lp["g1"] * _BN_C
        w1 = lp["W1"] * a1[None, :]
        b1 = (lp["b1"] * a1 + lp["be1"]).reshape(1, 2 * HID)
        a2 = lp["g2"] * _BN_C
        w2 = lp["W2"] * a2[None, :]
        b2 = (lp["b2"] * a2 + lp["be2"]).reshape(1, HID)
        onepe = (1.0 + lp["eps"]).astype(jnp.float32).reshape(1, 1)
        part = _sc_scatter(edges, h)
        if li < nlayers - 1:
            h = _mlp(onepe, h, part[0], part[1], w1, b1, w2, b2)
        else:
            pooled = _mlp_pool(onepe, h, part[0], part[1], w1, b1, w2, b2,
                               batch_r)

    a3 = params["g3"] * _BN_C
    l1w = params["lin1_W"] * a3[None, :]
    l1b = (params["lin1_b"] * a3 + params["be3"]).reshape(1, HID)
    l2b = params["lin2_b"].reshape(1, OUT)
    return _head(pooled, l1w, l1b, params["lin2_W"], l2b)


# speedup vs baseline: 2.9840x; 1.9803x over previous
import numpy as np
import jax
import jax.numpy as jnp
from jax import lax
from jax.experimental import pallas as pl
from jax.experimental.pallas import tpu as pltpu
from jax.experimental.pallas import tpu_sc as plsc

N = 10000
E = 320000
D = 128
DH = 64
HID = 128
OUT = 64
NGRAPH = 128

NCORES = 2
NSUB = 16
EPT = E // NSUB
CHUNK = 128
CHUNKS = 158
STAGES = 2
SCHUNKS = CHUNKS // STAGES
EPT_PAD = CHUNKS * CHUNK
NPAD = 10112
ZROWS = NPAD // NSUB

_BN_C = 1.0 / np.sqrt(1.0 + 1e-5)

BLK = 1000
NBLK = N // BLK


def _sc_body(edges, h, out, hsh, agg, src_buf, dst_buf, rows, sem):
    cid = lax.axis_index("c")
    sid = lax.axis_index("s")

    base = sid * ZROWS
    pltpu.sync_copy(h.at[cid, pl.ds(base, ZROWS)], hsh.at[pl.ds(base, ZROWS)])

    def _zrow(r, c):
        for k in range(4):
            rows[r, pl.ds(k * 16, 16)] = jnp.zeros((16,), jnp.float32)
        return c

    lax.fori_loop(0, CHUNK, _zrow, 0)
    pltpu.sync_copy(rows, agg.at[pl.ds(base, 128)])
    pltpu.sync_copy(rows, agg.at[pl.ds(base + 128, 128)])
    pltpu.sync_copy(rows, agg.at[pl.ds(base + 256, 128)])
    pltpu.sync_copy(rows, agg.at[pl.ds(base + 384, 128)])
    pltpu.sync_copy(rows.at[pl.ds(0, ZROWS - 512)],
                    agg.at[pl.ds(base + 512, ZROWS - 512)])
    plsc.subcore_barrier()

    for s in range(STAGES):
        pltpu.sync_copy(edges.at[0, sid, s], src_buf)
        pltpu.sync_copy(edges.at[1, sid, s], dst_buf)

        def _edge_chunk(j, c):
            pltpu.sync_copy(hsh.at[src_buf.at[j]], rows)
            pltpu.sync_copy(rows, agg.at[dst_buf.at[j]], add=True)
            return c

        lax.fori_loop(0, SCHUNKS, _edge_chunk, 0)
    plsc.subcore_barrier()

    pltpu.sync_copy(agg.at[pl.ds(base, ZROWS)],
                    out.at[cid, pl.ds(base, ZROWS)])


_sc_scatter = pl.kernel(
    _sc_body,
    out_type=jax.ShapeDtypeStruct((NCORES, NPAD, DH), jnp.float32),
    mesh=plsc.VectorSubcoreMesh(core_axis_name="c", subcore_axis_name="s",
                                num_cores=NCORES, num_subcores=NSUB),
    scratch_types=[
        pltpu.VMEM_SHARED((NPAD, DH), jnp.float32),
        pltpu.VMEM_SHARED((NPAD, DH), jnp.float32),
        pltpu.VMEM((SCHUNKS, CHUNK), jnp.int32),
        pltpu.VMEM((SCHUNKS, CHUNK), jnp.int32),
        pltpu.VMEM((CHUNK, DH), jnp.float32),
        pltpu.SemaphoreType.DMA,
    ],
)


def _mlp_math(eps_ref, h_ref, p_ref, w1_ref, b1_ref, w2_ref, b2_ref):
    z = (h_ref[0] * eps_ref[0, 0] + p_ref[0],
         h_ref[1] * eps_ref[0, 0] + p_ref[1])
    z = jnp.concatenate(z, axis=1)
    z = jnp.dot(z, w1_ref[...], preferred_element_type=jnp.float32) + b1_ref[...]
    z = jnp.maximum(z, 0.0)
    z = jnp.dot(z, w2_ref[...], preferred_element_type=jnp.float32) + b2_ref[...]
    return jnp.maximum(z, 0.0)


def _mlp_body(eps_ref, h_ref, p_ref, w1_ref, b1_ref, w2_ref, b2_ref, out_ref):
    hn = _mlp_math(eps_ref, h_ref, p_ref, w1_ref, b1_ref, w2_ref, b2_ref)
    out_ref[0] = hn[:, :DH]
    out_ref[1] = hn[:, DH:]


def _mlp_pool_body(eps_ref, h_ref, p_ref, w1_ref, b1_ref, w2_ref, b2_ref,
                   batch_ref, pooled_ref):
    hn = _mlp_math(eps_ref, h_ref, p_ref, w1_ref, b1_ref, w2_ref, b2_ref)
    seg = lax.broadcasted_iota(jnp.int32, (NGRAPH, BLK), 0)
    mask = (seg == batch_ref[0, 0, :][None, :]).astype(jnp.float32)
    contrib = jnp.dot(mask, hn, preferred_element_type=jnp.float32)
    i = pl.program_id(0)

    @pl.when(i == 0)
    def _():
        pooled_ref[...] = contrib

    @pl.when(i > 0)
    def _():
        pooled_ref[...] += contrib


def _head_body(pooled_ref, w1_ref, b1_ref, w2_ref, b2_ref, out_ref):
    z = jnp.dot(pooled_ref[...], w1_ref[...],
                preferred_element_type=jnp.float32) + b1_ref[...]
    z = jnp.maximum(z, 0.0)
    z = jnp.dot(z, w2_ref[...], preferred_element_type=jnp.float32) + b2_ref[...]
    m = jnp.max(z, axis=-1, keepdims=True)
    lse = jnp.log(jnp.sum(jnp.exp(z - m), axis=-1, keepdims=True)) + m
    out_ref[...] = z - lse


def _split_spec(i):
    return (0, i, 0)


def _fixed_spec(i):
    return (0, 0)


_COMMON_SPECS = [
    pl.BlockSpec(memory_space=pltpu.SMEM),
    pl.BlockSpec((2, BLK, DH), _split_spec),
    pl.BlockSpec((2, BLK, DH), _split_spec),
    pl.BlockSpec((D, 2 * HID), _fixed_spec),
    pl.BlockSpec((1, 2 * HID), _fixed_spec),
    pl.BlockSpec((2 * HID, HID), _fixed_spec),
    pl.BlockSpec((1, HID), _fixed_spec),
]

_mlp = pl.pallas_call(
    _mlp_body,
    grid=(NBLK,),
    in_specs=_COMMON_SPECS,
    out_specs=pl.BlockSpec((2, BLK, DH), _split_spec),
    out_shape=jax.ShapeDtypeStruct((2, NPAD, DH), jnp.float32),
)

_mlp_pool = pl.pallas_call(
    _mlp_pool_body,
    grid=(NBLK,),
    in_specs=_COMMON_SPECS + [pl.BlockSpec((1, 1, BLK), lambda i: (i, 0, 0))],
    out_specs=pl.BlockSpec((NGRAPH, HID), _fixed_spec),
    out_shape=jax.ShapeDtypeStruct((NGRAPH, HID), jnp.float32),
)

_head = pl.pallas_call(
    _head_body,
    out_shape=jax.ShapeDtypeStruct((NGRAPH, OUT), jnp.float32),
)


def kernel(x, edge_index, batch, params):
    pad = EPT_PAD - EPT
    src = jnp.concatenate(
        [edge_index[0].reshape(NSUB, EPT),
         jnp.zeros((NSUB, pad), jnp.int32)], axis=1)
    dst = jnp.concatenate(
        [edge_index[1].reshape(NSUB, EPT),
         jnp.full((NSUB, pad), N, jnp.int32)], axis=1)
    edges = jnp.stack([src, dst]).reshape(2, NSUB, STAGES, SCHUNKS, CHUNK)
    batch_r = batch.reshape(NBLK, 1, BLK)

    h = jnp.zeros((2, NPAD, DH), jnp.float32)
    h = h.at[0, :N].set(x[:, :DH]).at[1, :N].set(x[:, DH:])

    nlayers = len(params["layers"])
    pooled = None
    for li, lp in enumerate(params["layers"]):
        a1 = lp["g1"] * _BN_C
        w1 = lp["W1"] * a1[None, :]
        b1 = (lp["b1"] * a1 + lp["be1"]).reshape(1, 2 * HID)
        a2 = lp["g2"] * _BN_C
        w2 = lp["W2"] * a2[None, :]
        b2 = (lp["b2"] * a2 + lp["be2"]).reshape(1, HID)
        onepe = (1.0 + lp["eps"]).astype(jnp.float32).reshape(1, 1)
        part = _sc_scatter(edges, h)
        if li < nlayers - 1:
            h = _mlp(onepe, h, part, w1, b1, w2, b2)
        else:
            pooled = _mlp_pool(onepe, h, part, w1, b1, w2, b2, batch_r)

    a3 = params["g3"] * _BN_C
    l1w = params["lin1_W"] * a3[None, :]
    l1b = (params["lin1_b"] * a3 + params["be3"]).reshape(1, HID)
    l2b = params["lin2_b"].reshape(1, OUT)
    return _head(pooled, l1w, l1b, params["lin2_W"], l2b)
